# Initial kernel scaffold; baseline (speedup 1.0000x reference)
import jax
import jax.numpy as jnp
from jax.experimental import pallas as pl


def kernel(x, adj_edge_index, adj_values, W1, b1, W2, b2):
    raise NotImplementedError("write your pallas kernel here")


# speedup vs baseline: 29.2128x; 29.2128x over previous
import functools

import jax
import jax.numpy as jnp
from jax import lax
from jax.experimental import pallas as pl
from jax.experimental.pallas import tpu as pltpu
from jax.experimental.pallas import tpu_sc as plsc

_N = 10000
_E = 320000
_D = 128

_NC = 2
_NS = 16
_NW = _NC * _NS
_EPW = _E // _NW
_L = 16


def _dense_body(x_ref, w1_ref, b1_ref, w2_ref, s_ref):
    h = jnp.tanh(
        jax.lax.dot_general(
            x_ref[...], w1_ref[...], (((1,), (0,)), ((), ())),
            preferred_element_type=jnp.float32,
        )
        + b1_ref[...]
    )
    s_ref[...] = jax.lax.dot_general(
        h, w2_ref[...], (((1,), (0,)), ((), ())),
        preferred_element_type=jnp.float32,
    )


def _dense_call(x, W1, b1_2d, W2):
    blk = 2000
    return pl.pallas_call(
        _dense_body,
        grid=(_N // blk,),
        in_specs=[
            pl.BlockSpec((blk, _D), lambda i: (i, 0)),
            pl.BlockSpec((_D, _D), lambda i: (0, 0)),
            pl.BlockSpec((1, _D), lambda i: (0, 0)),
            pl.BlockSpec((_D, 1), lambda i: (0, 0)),
        ],
        out_specs=pl.BlockSpec((blk, 1), lambda i: (i, 0)),
        out_shape=jax.ShapeDtypeStruct((_N, 1), jnp.float32),
    )(x, W1, b1_2d, W2)


_sc_mesh = plsc.VectorSubcoreMesh(core_axis_name="c", subcore_axis_name="s")


@functools.partial(
    pl.kernel,
    out_type=jax.ShapeDtypeStruct((_NW, _N), jnp.float32),
    mesh=_sc_mesh,
    scratch_types=[
        pltpu.VMEM((_N,), jnp.float32),
        pltpu.VMEM((_EPW,), jnp.int32),
        pltpu.VMEM((_EPW,), jnp.int32),
        pltpu.VMEM((_EPW,), jnp.float32),
        pltpu.VMEM((_N,), jnp.float32),
    ],
    compiler_params=pltpu.CompilerParams(needs_layout_passes=False),
)
def _sparse_kernel(s_hbm, row_hbm, col_hbm, val_hbm, out_hbm,
                   s_v, row_v, col_v, val_v, acc_v):
    cid = lax.axis_index("c")
    sid = lax.axis_index("s")
    wid = sid * _NC + cid
    base = wid * _EPW

    pltpu.sync_copy(s_hbm, s_v)
    pltpu.sync_copy(row_hbm.at[pl.ds(base, _EPW)], row_v)
    pltpu.sync_copy(col_hbm.at[pl.ds(base, _EPW)], col_v)
    pltpu.sync_copy(val_hbm.at[pl.ds(base, _EPW)], val_v)

    def _zero(i, carry):
        acc_v[pl.ds(i * _L, _L)] = jnp.zeros((_L,), jnp.float32)
        return carry

    lax.fori_loop(0, _N // _L, _zero, 0)

    def _edge(i, carry):
        off = i * _L
        r = row_v[pl.ds(off, _L)]
        c = col_v[pl.ds(off, _L)]
        v = val_v[pl.ds(off, _L)]
        g = plsc.load_gather(s_v, [c])
        plsc.addupdate_scatter(acc_v, [r], g * v)
        return carry

    lax.fori_loop(0, _EPW // _L, _edge, 0)

    pltpu.sync_copy(acc_v, out_hbm.at[wid])


def _reduce_body(p_ref, b2_ref, o_ref):
    o_ref[...] = jnp.sum(p_ref[...], axis=0, keepdims=True) + b2_ref[...]


def _reduce_call(partials, b2_2d):
    return pl.pallas_call(
        _reduce_body,
        in_specs=[
            pl.BlockSpec((_NW, _N), lambda: (0, 0)),
            pl.BlockSpec((1, 1), lambda: (0, 0)),
        ],
        out_specs=pl.BlockSpec((1, _N), lambda: (0, 0)),
        out_shape=jax.ShapeDtypeStruct((1, _N), jnp.float32),
    )(partials, b2_2d)


def kernel(x, adj_edge_index, adj_values, W1, b1, W2, b2):
    s = _dense_call(x, W1, b1.reshape(1, _D), W2)
    row = adj_edge_index[0]
    col = adj_edge_index[1]
    partials = _sparse_kernel(s.reshape(_N), row, col, adj_values)
    out = _reduce_call(partials, b2.reshape(1, 1))
    return out.reshape(_N, 1)


# speedup vs baseline: 41.4407x; 1.4186x over previous
import functools

import jax
import jax.numpy as jnp
from jax import lax
from jax.experimental import pallas as pl
from jax.experimental.pallas import tpu as pltpu
from jax.experimental.pallas import tpu_sc as plsc

_N = 10000
_E = 320000
_D = 128

_NC = 2
_NS = 16
_NW = _NC * _NS
_EPW = _E // _NW
_L = 16


def _dense_body(x_ref, w1_ref, b1_ref, w2_ref, s_ref):
    h = jnp.tanh(
        jax.lax.dot_general(
            x_ref[...], w1_ref[...], (((1,), (0,)), ((), ())),
            preferred_element_type=jnp.float32,
        )
        + b1_ref[...]
    )
    s_ref[...] = jax.lax.dot_general(
        h, w2_ref[...], (((1,), (0,)), ((), ())),
        preferred_element_type=jnp.float32,
    )


def _dense_call(x, W1, b1_2d, W2):
    blk = 2000
    return pl.pallas_call(
        _dense_body,
        grid=(_N // blk,),
        in_specs=[
            pl.BlockSpec((blk, _D), lambda i: (i, 0)),
            pl.BlockSpec((_D, _D), lambda i: (0, 0)),
            pl.BlockSpec((1, _D), lambda i: (0, 0)),
            pl.BlockSpec((_D, 1), lambda i: (0, 0)),
        ],
        out_specs=pl.BlockSpec((blk, 1), lambda i: (i, 0)),
        out_shape=jax.ShapeDtypeStruct((_N, 1), jnp.float32),
    )(x, W1, b1_2d, W2)


_sc_mesh = plsc.VectorSubcoreMesh(core_axis_name="c", subcore_axis_name="s")


@functools.partial(
    pl.kernel,
    out_type=jax.ShapeDtypeStruct((_NW, _N), jnp.float32),
    mesh=_sc_mesh,
    scratch_types=[
        pltpu.VMEM((_N,), jnp.float32),
        pltpu.VMEM((_EPW,), jnp.int32),
        pltpu.VMEM((_EPW,), jnp.int32),
        pltpu.VMEM((_EPW,), jnp.float32),
        pltpu.VMEM((_N,), jnp.float32),
        pltpu.SemaphoreType.DMA,
        pltpu.SemaphoreType.DMA,
        pltpu.SemaphoreType.DMA,
        pltpu.SemaphoreType.DMA,
    ],
    compiler_params=pltpu.CompilerParams(needs_layout_passes=False),
)
def _sparse_kernel(s_hbm, ei_hbm, val_hbm, out_hbm,
                   s_v, row_v, col_v, val_v, acc_v,
                   sem0, sem1, sem2, sem3):
    cid = lax.axis_index("c")
    sid = lax.axis_index("s")
    wid = sid * _NC + cid
    base = wid * _EPW

    cp0 = pltpu.async_copy(s_hbm, s_v, sem0)
    cp1 = pltpu.async_copy(ei_hbm.at[pl.ds(base, _EPW)], row_v, sem1)
    cp2 = pltpu.async_copy(ei_hbm.at[pl.ds(_E + base, _EPW)], col_v, sem2)
    cp3 = pltpu.async_copy(val_hbm.at[pl.ds(base, _EPW)], val_v, sem3)

    @plsc.parallel_loop(0, _N // _L, unroll=5)
    def _zero(i):
        acc_v[pl.ds(i * _L, _L)] = jnp.zeros((_L,), jnp.float32)

    cp0.wait()
    cp1.wait()
    cp2.wait()
    cp3.wait()

    @plsc.parallel_loop(0, _EPW // _L, unroll=5)
    def _edge(i):
        off = i * _L
        r = row_v[pl.ds(off, _L)]
        c = col_v[pl.ds(off, _L)]
        v = val_v[pl.ds(off, _L)]
        g = plsc.load_gather(s_v, [c])
        plsc.addupdate_scatter(acc_v, [r], g * v)

    pltpu.sync_copy(acc_v, out_hbm.at[wid])


def _reduce_body(p_ref, b2_ref, o_ref):
    o_ref[...] = jnp.sum(p_ref[...], axis=0, keepdims=True) + b2_ref[...]


def _reduce_call(partials, b2_2d):
    return pl.pallas_call(
        _reduce_body,
        in_specs=[
            pl.BlockSpec((_NW, _N), lambda: (0, 0)),
            pl.BlockSpec((1, 1), lambda: (0, 0)),
        ],
        out_specs=pl.BlockSpec((1, _N), lambda: (0, 0)),
        out_shape=jax.ShapeDtypeStruct((1, _N), jnp.float32),
    )(partials, b2_2d)


def kernel(x, adj_edge_index, adj_values, W1, b1, W2, b2):
    s = _dense_call(x, W1, b1.reshape(1, _D), W2)
    partials = _sparse_kernel(s.reshape(_N), adj_edge_index.reshape(2 * _E),
                              adj_values)
    out = _reduce_call(partials, b2.reshape(1, 1))
    return out.reshape(_N, 1)


# speedup vs baseline: 48.7547x; 1.1765x over previous
import functools

import jax
import jax.numpy as jnp
from jax import lax
from jax.experimental import pallas as pl
from jax.experimental.pallas import tpu as pltpu
from jax.experimental.pallas import tpu_sc as plsc

_N = 10000
_E = 320000
_D = 128

_NC = 2
_NS = 16
_NW = _NC * _NS
_L = 16

_CK = 128
_NCHUNK = _E // _CK
_MAXSPAN = ((_NCHUNK + _NW - 1) // _NW) * _CK


def _dense_body(x_ref, w1_ref, b1_ref, w2_ref, s_ref):
    h = jnp.tanh(
        lax.dot_general(
            x_ref[...], w1_ref[...], (((1,), (0,)), ((), ())),
            preferred_element_type=jnp.float32,
        )
        + b1_ref[...]
    )
    s_ref[...] = lax.dot_general(
        w2_ref[...], h, (((1,), (1,)), ((), ())),
        preferred_element_type=jnp.float32,
    )


_NP = 10240


def _dense_call(x, W1, b1_2d, w2_row):
    blk = 2048
    return pl.pallas_call(
        _dense_body,
        grid=(_NP // blk,),
        in_specs=[
            pl.BlockSpec((blk, _D), lambda i: (i, 0)),
            pl.BlockSpec((_D, _D), lambda i: (0, 0)),
            pl.BlockSpec((1, _D), lambda i: (0, 0)),
            pl.BlockSpec((1, _D), lambda i: (0, 0)),
        ],
        out_specs=pl.BlockSpec((1, blk), lambda i: (0, i)),
        out_shape=jax.ShapeDtypeStruct((1, _NP), jnp.float32),
    )(x, W1, b1_2d, w2_row)


_sc_mesh = plsc.VectorSubcoreMesh(core_axis_name="c", subcore_axis_name="s")


@functools.partial(
    pl.kernel,
    out_type=jax.ShapeDtypeStruct((_NW, _N), jnp.float32),
    mesh=_sc_mesh,
    scratch_types=[
        pltpu.VMEM((_NP,), jnp.float32),
        pltpu.VMEM((2, _MAXSPAN), jnp.int32),
        pltpu.VMEM((_MAXSPAN,), jnp.float32),
        pltpu.VMEM((_N,), jnp.float32),
        pltpu.SemaphoreType.DMA,
        pltpu.SemaphoreType.DMA,
        pltpu.SemaphoreType.DMA,
    ],
    compiler_params=pltpu.CompilerParams(needs_layout_passes=False),
)
def _sparse_kernel(s_hbm, ei_hbm, val_hbm, out_hbm,
                   s_v, ei_v, val_v, acc_v, sem0, sem1, sem2):
    cid = lax.axis_index("c")
    sid = lax.axis_index("s")
    wid = sid * _NC + cid
    start = (625 * wid) >> 3
    end = (625 * (wid + 1)) >> 3
    n16 = (end - start) * (_CK // _L)
    base = start * _CK

    cp0 = pltpu.async_copy(s_hbm.at[0], s_v, sem0)
    cp1 = pltpu.async_copy(ei_hbm.at[:, pl.ds(base, _MAXSPAN)], ei_v, sem1)
    cp2 = pltpu.async_copy(val_hbm.at[pl.ds(base, _MAXSPAN)], val_v, sem2)

    @plsc.parallel_loop(0, _N // _L, unroll=5)
    def _zero(i):
        acc_v[pl.ds(i * _L, _L)] = jnp.zeros((_L,), jnp.float32)

    cp0.wait()
    cp1.wait()
    cp2.wait()

    @plsc.parallel_loop(0, n16, unroll=4)
    def _edge(i):
        off = i * _L
        r = ei_v[0, pl.ds(off, _L)]
        c = ei_v[1, pl.ds(off, _L)]
        v = val_v[pl.ds(off, _L)]
        g = plsc.load_gather(s_v, [c])
        plsc.addupdate_scatter(acc_v, [r], g * v)

    pltpu.sync_copy(acc_v, out_hbm.at[wid])


def _reduce_body(p_ref, b2_ref, o_ref):
    ones = jnp.ones((_NW, 1), jnp.float32)
    o_ref[...] = lax.dot_general(
        p_ref[...], ones, (((0,), (0,)), ((), ())),
        preferred_element_type=jnp.float32,
    ) + b2_ref[...]


def _reduce_call(partials, b2_2d):
    return pl.pallas_call(
        _reduce_body,
        in_specs=[
            pl.BlockSpec((_NW, _N), lambda: (0, 0)),
            pl.BlockSpec((1, 1), lambda: (0, 0)),
        ],
        out_specs=pl.BlockSpec((_N, 1), lambda: (0, 0)),
        out_shape=jax.ShapeDtypeStruct((_N, 1), jnp.float32),
    )(partials, b2_2d)


def kernel(x, adj_edge_index, adj_values, W1, b1, W2, b2):
    s = _dense_call(x, W1, b1.reshape(1, _D), W2.reshape(1, _D))
    partials = _sparse_kernel(s, adj_edge_index, adj_values)
    return _reduce_call(partials, b2.reshape(1, 1))


# speedup vs baseline: 54.6144x; 1.1202x over previous
import functools

import jax
import jax.numpy as jnp
from jax import lax
from jax.experimental import pallas as pl
from jax.experimental.pallas import tpu as pltpu
from jax.experimental.pallas import tpu_sc as plsc

_N = 10000
_E = 320000
_D = 128

_NC = 2
_NS = 16
_NW = _NC * _NS
_L = 16

_CK = 128
_NCHUNK = _E // _CK
_MAXSPAN = ((_NCHUNK + _NW - 1) // _NW) * _CK


def _dense_body(x_ref, w1_ref, b1_ref, w2_ref, s_ref):
    h = jnp.tanh(
        lax.dot_general(
            x_ref[...], w1_ref[...], (((1,), (0,)), ((), ())),
            preferred_element_type=jnp.float32,
        )
        + b1_ref[...]
    )
    s_ref[...] = lax.dot_general(
        w2_ref[...], h, (((1,), (1,)), ((), ())),
        preferred_element_type=jnp.float32,
    )


_NP = 10240


def _dense_call(x, W1, b1_2d, w2_row):
    blk = 1024
    return pl.pallas_call(
        _dense_body,
        grid=(_NP // blk,),
        in_specs=[
            pl.BlockSpec((blk, _D), lambda i: (i, 0)),
            pl.BlockSpec((_D, _D), lambda i: (0, 0)),
            pl.BlockSpec((1, _D), lambda i: (0, 0)),
            pl.BlockSpec((1, _D), lambda i: (0, 0)),
        ],
        out_specs=pl.BlockSpec((1, blk), lambda i: (0, i)),
        out_shape=jax.ShapeDtypeStruct((1, _NP), jnp.float32),
    )(x, W1, b1_2d, w2_row)


_sc_mesh = plsc.VectorSubcoreMesh(core_axis_name="c", subcore_axis_name="s")


@functools.partial(
    pl.kernel,
    out_type=jax.ShapeDtypeStruct((_NW, _N), jnp.float32),
    mesh=_sc_mesh,
    scratch_types=[
        pltpu.VMEM((_NP,), jnp.float32),
        pltpu.VMEM((2, _MAXSPAN), jnp.int32),
        pltpu.VMEM((_MAXSPAN,), jnp.float32),
        pltpu.VMEM((_N,), jnp.float32),
        pltpu.SemaphoreType.DMA,
        pltpu.SemaphoreType.DMA,
        pltpu.SemaphoreType.DMA,
    ],
    compiler_params=pltpu.CompilerParams(needs_layout_passes=False),
)
def _sparse_kernel(s_hbm, ei_hbm, val_hbm, out_hbm,
                   s_v, ei_v, val_v, acc_v, sem0, sem1, sem2):
    cid = lax.axis_index("c")
    sid = lax.axis_index("s")
    wid = sid * _NC + cid
    start = (625 * wid) >> 3
    end = (625 * (wid + 1)) >> 3
    n16 = (end - start) * (_CK // _L)
    base = start * _CK

    cp0 = pltpu.async_copy(s_hbm.at[0], s_v, sem0)
    cp1 = pltpu.async_copy(ei_hbm.at[:, pl.ds(base, _MAXSPAN)], ei_v, sem1)
    cp2 = pltpu.async_copy(val_hbm.at[pl.ds(base, _MAXSPAN)], val_v, sem2)

    @plsc.parallel_loop(0, _N // _L, unroll=5)
    def _zero(i):
        acc_v[pl.ds(i * _L, _L)] = jnp.zeros((_L,), jnp.float32)

    cp0.wait()
    cp1.wait()
    cp2.wait()

    @plsc.parallel_loop(0, n16, unroll=8)
    def _edge(i):
        off = i * _L
        r = ei_v[0, pl.ds(off, _L)]
        c = ei_v[1, pl.ds(off, _L)]
        v = val_v[pl.ds(off, _L)]
        g = plsc.load_gather(s_v, [c])
        plsc.addupdate_scatter(acc_v, [r], g * v)

    pltpu.sync_copy(acc_v, out_hbm.at[wid])


def _reduce_body(p_ref, b2_ref, o_ref):
    o_ref[...] = jnp.sum(p_ref[...], axis=0, keepdims=True) + b2_ref[...]


def _reduce_call(partials, b2_2d):
    return pl.pallas_call(
        _reduce_body,
        in_specs=[
            pl.BlockSpec((_NW, _N), lambda: (0, 0)),
            pl.BlockSpec((1, 1), lambda: (0, 0)),
        ],
        out_specs=pl.BlockSpec((1, _N), lambda: (0, 0)),
        out_shape=jax.ShapeDtypeStruct((1, _N), jnp.float32),
    )(partials, b2_2d)


def kernel(x, adj_edge_index, adj_values, W1, b1, W2, b2):
    s = _dense_call(x, W1, b1.reshape(1, _D), W2.reshape(1, _D))
    partials = _sparse_kernel(s, adj_edge_index, adj_values)
    out = _reduce_call(partials, b2.reshape(1, 1))
    return out.reshape(_N, 1)


# speedup vs baseline: 58.7893x; 1.0764x over previous
import functools

import jax
import jax.numpy as jnp
from jax import lax
from jax.experimental import pallas as pl
from jax.experimental.pallas import tpu as pltpu
from jax.experimental.pallas import tpu_sc as plsc

_N = 10000
_E = 320000
_D = 128

_NC = 2
_NS = 16
_NW = _NC * _NS
_L = 16

_CK = 128
_NCHUNK = _E // _CK
_MAXSPAN = ((_NCHUNK + _NW - 1) // _NW) * _CK


def _dense_body(x_ref, w1_ref, b1_ref, w2_ref, s_ref):
    h = jnp.tanh(
        lax.dot_general(
            x_ref[...], w1_ref[...], (((1,), (0,)), ((), ())),
            preferred_element_type=jnp.float32,
        )
        + b1_ref[...]
    )
    s_ref[...] = lax.dot_general(
        w2_ref[...], h, (((1,), (1,)), ((), ())),
        preferred_element_type=jnp.float32,
    )


_NP = 10240


def _dense_call(x, W1, b1_2d, w2_row):
    blk = 2048
    return pl.pallas_call(
        _dense_body,
        grid=(_NP // blk,),
        in_specs=[
            pl.BlockSpec((blk, _D), lambda i: (i, 0)),
            pl.BlockSpec((_D, _D), lambda i: (0, 0)),
            pl.BlockSpec((1, _D), lambda i: (0, 0)),
            pl.BlockSpec((1, _D), lambda i: (0, 0)),
        ],
        out_specs=pl.BlockSpec((1, blk), lambda i: (0, i)),
        out_shape=jax.ShapeDtypeStruct((1, _NP), jnp.float32),
    )(x, W1, b1_2d, w2_row)


_sc_mesh = plsc.VectorSubcoreMesh(core_axis_name="c", subcore_axis_name="s")


@functools.partial(
    pl.kernel,
    out_type=jax.ShapeDtypeStruct((_NW, _N), jnp.float32),
    mesh=_sc_mesh,
    scratch_types=[
        pltpu.VMEM((_NP,), jnp.float32),
        pltpu.VMEM((2, _MAXSPAN), jnp.int32),
        pltpu.VMEM((_MAXSPAN,), jnp.float32),
        pltpu.VMEM((_N,), jnp.float32),
        pltpu.SemaphoreType.DMA,
        pltpu.SemaphoreType.DMA,
        pltpu.SemaphoreType.DMA,
    ],
    compiler_params=pltpu.CompilerParams(needs_layout_passes=False),
)
def _sparse_kernel(s_hbm, ei_hbm, val_hbm, out_hbm,
                   s_v, ei_v, val_v, acc_v, sem0, sem1, sem2):
    cid = lax.axis_index("c")
    sid = lax.axis_index("s")
    wid = sid * _NC + cid
    start = (625 * wid) >> 3
    end = (625 * (wid + 1)) >> 3
    n16 = (end - start) * (_CK // _L)
    base = start * _CK

    cp0 = pltpu.async_copy(s_hbm.at[0], s_v, sem0)
    cp1 = pltpu.async_copy(ei_hbm.at[:, pl.ds(base, _MAXSPAN)], ei_v, sem1)
    cp2 = pltpu.async_copy(val_hbm.at[pl.ds(base, _MAXSPAN)], val_v, sem2)

    @plsc.parallel_loop(0, _N // _L, unroll=5)
    def _zero(i):
        acc_v[pl.ds(i * _L, _L)] = jnp.zeros((_L,), jnp.float32)

    cp0.wait()
    cp1.wait()
    cp2.wait()

    @plsc.parallel_loop(0, n16, unroll=12)
    def _edge(i):
        off = i * _L
        r = ei_v[0, pl.ds(off, _L)]
        c = ei_v[1, pl.ds(off, _L)]
        v = val_v[pl.ds(off, _L)]
        g = plsc.load_gather(s_v, [c])
        plsc.addupdate_scatter(acc_v, [r], g * v)

    pltpu.sync_copy(acc_v, out_hbm.at[wid])


def _reduce_body(p_ref, b2_ref, o_ref):
    o_ref[...] = jnp.sum(p_ref[...], axis=0, keepdims=True) + b2_ref[...]


def _reduce_call(partials, b2_2d):
    return pl.pallas_call(
        _reduce_body,
        in_specs=[
            pl.BlockSpec((_NW, _N), lambda: (0, 0)),
            pl.BlockSpec((1, 1), lambda: (0, 0)),
        ],
        out_specs=pl.BlockSpec((1, _N), lambda: (0, 0)),
        out_shape=jax.ShapeDtypeStruct((1, _N), jnp.float32),
    )(partials, b2_2d)


def kernel(x, adj_edge_index, adj_values, W1, b1, W2, b2):
    s = _dense_call(x, W1, b1.reshape(1, _D), W2.reshape(1, _D))
    partials = _sparse_kernel(s, adj_edge_index, adj_values)
    out = _reduce_call(partials, b2.reshape(1, 1))
    return out.reshape(_N, 1)
